# baseline (speedup 1.0000x reference)
import functools

import jax
import jax.numpy as jnp
from jax import lax
from jax.experimental import pallas as pl
from jax.experimental.pallas import tpu as pltpu
from jax.experimental.pallas import tpu_sc as plsc

_B, _C1, _HW = 16, 3, 224
_E, _TOPK = 8, 2
_C2 = 96
_H56 = 56
_NPAD = _H56 * _H56
_HO = 55
_NF = _HO * _H56
_K = 192
_SHIFTS = (0, 1, _H56, _H56 + 1)
_NPOOL = _HW * _HW


def _pool_body(x_ref, wgt_ref, logits_ref):
    scale = 1.0 / _NPOOL
    rows = jnp.sum(x_ref[...], axis=2)
    acc = None
    for c in range(_C1):
        s = jnp.sum(rows[:, c * _HW:(c + 1) * _HW], axis=1)
        t = wgt_ref[:, c:c + 1] * s.reshape(1, _B)
        acc = t if acc is None else acc + t
    logits_ref[...] = acc * scale


def _route_body(logits_hbm, idx_hbm, gates_hbm, logits_v, idx_v, gates_v):
    wid = lax.axis_index("s") * 2 + lax.axis_index("c")

    @pl.when(wid == 0)
    def _():
        pltpu.sync_copy(logits_hbm, logits_v)
        neg = jnp.full((16,), -3e38, jnp.float32)
        m1, m2 = neg, neg
        i1 = jnp.zeros((16,), jnp.int32)
        i2 = jnp.zeros((16,), jnp.int32)
        for e in range(_E):
            v = logits_v[e, :]
            ei = jnp.full((16,), e, jnp.int32)
            gt1 = v > m1
            gt2 = v > m2
            m2 = jnp.where(gt1, m1, jnp.where(gt2, v, m2))
            i2 = jnp.where(gt1, i1, jnp.where(gt2, ei, i2))
            m1 = jnp.where(gt1, v, m1)
            i1 = jnp.where(gt1, ei, i1)
        d = jnp.exp(m2 - m1)
        den = 1.0 + d
        g1 = 1.0 / den
        g2 = d / den
        idx_v[pl.ds(0, 16)] = i1
        idx_v[pl.ds(16, 16)] = i2
        gates_v[pl.ds(0, 16)] = g1
        gates_v[pl.ds(16, 16)] = g2
        pltpu.sync_copy(idx_v, idx_hbm)
        pltpu.sync_copy(gates_v, gates_hbm)


_TILE = 448
_NT = 7


def _moe_body(idx_ref, g_ref, xs_ref, w1_ref, w2_ref, p1_ref, p2_ref,
              out_ref, rhs):
    b = pl.program_id(0)
    wcat = jnp.concatenate([w1_ref[0], w2_ref[0]], axis=0)
    for t0 in range(_NT):
        c0 = t0 * _TILE
        w = min(_NF - c0, _TILE)
        for t, s in enumerate(_SHIFTS):
            wr = min(w, _NPAD - c0 - s)
            rhs[t * 48:(t + 1) * 48, :wr] = xs_ref[0, :, c0 + s:c0 + s + wr]
        acc = lax.dot_general(wcat, rhs[:, :w], (((1,), (0,)), ((), ())),
                              preferred_element_type=jnp.float32)
        out = None
        for h, (p_ref, goff) in enumerate(((p1_ref, 0), (p2_ref, _B))):
            y = acc[h * _C2:(h + 1) * _C2, :] + p_ref[0, :, 0:1]
            u = jnp.mean(y, axis=0, keepdims=True)
            dlt = y - u
            var = jnp.mean(dlt * dlt, axis=0, keepdims=True)
            yn = dlt * lax.rsqrt(var + 1e-6)
            yn = yn * p_ref[0, :, 1:2] + p_ref[0, :, 2:3]
            gate = g_ref[goff + b]
            out = yn * gate if out is None else out + yn * gate
        nrows = 8 if t0 < _NT - 1 else 7
        out3 = out.reshape(_C2, nrows, _H56)
        out_ref[0, :, t0 * 8:t0 * 8 + nrows, :] = out3[:, :, :_HO]


def _pool_call(x3, w_gate):
    return pl.pallas_call(
        _pool_body,
        out_shape=jax.ShapeDtypeStruct((_E, _B), jnp.float32),
        in_specs=[
            pl.BlockSpec((_B, _C1 * _HW, _HW), lambda: (0, 0, 0)),
            pl.BlockSpec((_E, _C1), lambda: (0, 0)),
        ],
        out_specs=pl.BlockSpec((_E, _B), lambda: (0, 0)),
    )(x3, w_gate)


@functools.cache
def _route_call():
    return functools.partial(
        pl.kernel,
        out_type=(jax.ShapeDtypeStruct((2 * _B,), jnp.int32),
                  jax.ShapeDtypeStruct((2 * _B,), jnp.float32)),
        mesh=plsc.VectorSubcoreMesh(core_axis_name="c", subcore_axis_name="s"),
        scratch_types=[
            pltpu.VMEM((_E, _B), jnp.float32),
            pltpu.VMEM((2 * _B,), jnp.int32),
            pltpu.VMEM((2 * _B,), jnp.float32),
        ],
    )(_route_body)


def _moe_call(top_idx, gates, xs2, wt, prm):
    grid_spec = pltpu.PrefetchScalarGridSpec(
        num_scalar_prefetch=2,
        grid=(_B,),
        in_specs=[
            pl.BlockSpec((1, 48, _NPAD), lambda b, idx, g: (b, 0, 0)),
            pl.BlockSpec((1, _C2, _K), lambda b, idx, g: (idx[b], 0, 0)),
            pl.BlockSpec((1, _C2, _K), lambda b, idx, g: (idx[_B + b], 0, 0)),
            pl.BlockSpec((1, _C2, 3), lambda b, idx, g: (idx[b], 0, 0)),
            pl.BlockSpec((1, _C2, 3), lambda b, idx, g: (idx[_B + b], 0, 0)),
        ],
        out_specs=pl.BlockSpec((1, _C2, _HO, _HO),
                               lambda b, idx, g: (b, 0, 0, 0)),
        scratch_shapes=[pltpu.VMEM((_K, _TILE), jnp.float32)],
    )
    return pl.pallas_call(
        _moe_body,
        grid_spec=grid_spec,
        out_shape=jax.ShapeDtypeStruct((_B, _C2, _HO, _HO), jnp.float32),
        compiler_params=pltpu.CompilerParams(
            dimension_semantics=("arbitrary",)),
    )(top_idx, gates, xs2, wt, wt, prm, prm)


def kernel(x, w_gate, proj_w, proj_b, ln_w, ln_b):
    xr = x.reshape(_B, _C1, _H56, 4, _H56, 4)
    xr = xr.transpose(0, 1, 3, 5, 2, 4)
    xs2 = xr.reshape(_B, 48, _NPAD)

    wp = jnp.pad(proj_w, ((0, 0), (0, 0), (0, 0), (0, 1), (0, 1)))
    wt = wp.reshape(_E, _C2, _C1, 2, 4, 2, 4)
    wt = wt.transpose(0, 1, 3, 5, 2, 4, 6).reshape(_E, _C2, _K)

    prm = jnp.stack([proj_b, ln_w, ln_b], axis=-1)

    logits = _pool_call(x.reshape(_B, _C1 * _HW, _HW), w_gate.T)
    top_idx, gates = _route_call()(logits)
    return _moe_call(top_idx, gates, xs2, wt, prm)


# speedup vs baseline: 1.1804x; 1.1804x over previous
import functools

import jax
import jax.numpy as jnp
from jax import lax
from jax.experimental import pallas as pl
from jax.experimental.pallas import tpu as pltpu
from jax.experimental.pallas import tpu_sc as plsc

_B, _C1, _HW = 16, 3, 224
_E, _TOPK = 8, 2
_C2 = 96
_H56 = 56
_NPAD = _H56 * _H56
_HO = 55
_NF = _HO * _H56
_K = 192
_SHIFTS = (0, 1, _H56, _H56 + 1)
_NPOOL = _HW * _HW


def _pool_body(x_ref, wgt_ref, logits_ref):
    scale = 1.0 / _NPOOL
    rows = jnp.sum(x_ref[...], axis=2)
    acc = None
    for c in range(_C1):
        s = jnp.sum(rows[:, c * _HW:(c + 1) * _HW], axis=1)
        t = wgt_ref[:, c:c + 1] * s.reshape(1, _B)
        acc = t if acc is None else acc + t
    logits_ref[...] = acc * scale


def _route_body(logits_hbm, idx_hbm, gates_hbm, logits_v, idx_v, gates_v):
    wid = lax.axis_index("s") * 2 + lax.axis_index("c")

    @pl.when(wid == 0)
    def _():
        pltpu.sync_copy(logits_hbm, logits_v)
        neg = jnp.full((16,), -3e38, jnp.float32)
        m1, m2 = neg, neg
        i1 = jnp.zeros((16,), jnp.int32)
        i2 = jnp.zeros((16,), jnp.int32)
        for e in range(_E):
            v = logits_v[e, :]
            ei = jnp.full((16,), e, jnp.int32)
            gt1 = v > m1
            gt2 = v > m2
            m2 = jnp.where(gt1, m1, jnp.where(gt2, v, m2))
            i2 = jnp.where(gt1, i1, jnp.where(gt2, ei, i2))
            m1 = jnp.where(gt1, v, m1)
            i1 = jnp.where(gt1, ei, i1)
        d = jnp.exp(m2 - m1)
        den = 1.0 + d
        g1 = 1.0 / den
        g2 = d / den
        idx_v[pl.ds(0, 16)] = i1
        idx_v[pl.ds(16, 16)] = i2
        gates_v[pl.ds(0, 16)] = g1
        gates_v[pl.ds(16, 16)] = g2
        pltpu.sync_copy(idx_v, idx_hbm)
        pltpu.sync_copy(gates_v, gates_hbm)


_TILE = 896
_NT = 4


def _moe_body(idx_ref, g_ref, xs_ref, w1_ref, w2_ref, p1_ref, p2_ref,
              out_ref, rhs):
    b = pl.program_id(0)
    wcat = jnp.concatenate([w1_ref[0], w2_ref[0]], axis=0)
    for t0 in range(_NT):
        c0 = t0 * _TILE
        w = min(_NF - c0, _TILE)
        for t, s in enumerate(_SHIFTS):
            wr = min(w, _NPAD - c0 - s)
            rhs[t * 48:(t + 1) * 48, :wr] = xs_ref[0, :, c0 + s:c0 + s + wr]
        acc = lax.dot_general(wcat, rhs[:, :w], (((1,), (0,)), ((), ())),
                              preferred_element_type=jnp.float32)
        out = None
        for h, (p_ref, goff) in enumerate(((p1_ref, 0), (p2_ref, _B))):
            y = acc[h * _C2:(h + 1) * _C2, :] + p_ref[0, :, 0:1]
            u = jnp.mean(y, axis=0, keepdims=True)
            dlt = y - u
            var = jnp.mean(dlt * dlt, axis=0, keepdims=True)
            yn = dlt * lax.rsqrt(var + 1e-6)
            yn = yn * p_ref[0, :, 1:2] + p_ref[0, :, 2:3]
            gate = g_ref[goff + b]
            out = yn * gate if out is None else out + yn * gate
        out_ref[0, :, c0:c0 + w] = out


def _pool_call(x3, w_gate):
    return pl.pallas_call(
        _pool_body,
        out_shape=jax.ShapeDtypeStruct((_E, _B), jnp.float32),
        in_specs=[
            pl.BlockSpec((_B, _C1 * _HW, _HW), lambda: (0, 0, 0)),
            pl.BlockSpec((_E, _C1), lambda: (0, 0)),
        ],
        out_specs=pl.BlockSpec((_E, _B), lambda: (0, 0)),
    )(x3, w_gate)


@functools.cache
def _route_call():
    return functools.partial(
        pl.kernel,
        out_type=(jax.ShapeDtypeStruct((2 * _B,), jnp.int32),
                  jax.ShapeDtypeStruct((2 * _B,), jnp.float32)),
        mesh=plsc.VectorSubcoreMesh(core_axis_name="c", subcore_axis_name="s"),
        scratch_types=[
            pltpu.VMEM((_E, _B), jnp.float32),
            pltpu.VMEM((2 * _B,), jnp.int32),
            pltpu.VMEM((2 * _B,), jnp.float32),
        ],
    )(_route_body)


def _moe_call(top_idx, gates, xs2, wt, prm):
    grid_spec = pltpu.PrefetchScalarGridSpec(
        num_scalar_prefetch=2,
        grid=(_B,),
        in_specs=[
            pl.BlockSpec((1, 48, _NPAD), lambda b, idx, g: (b, 0, 0)),
            pl.BlockSpec((1, _C2, _K), lambda b, idx, g: (idx[b], 0, 0)),
            pl.BlockSpec((1, _C2, _K), lambda b, idx, g: (idx[_B + b], 0, 0)),
            pl.BlockSpec((1, _C2, 3), lambda b, idx, g: (idx[b], 0, 0)),
            pl.BlockSpec((1, _C2, 3), lambda b, idx, g: (idx[_B + b], 0, 0)),
        ],
        out_specs=pl.BlockSpec((1, _C2, _NF), lambda b, idx, g: (b, 0, 0)),
        scratch_shapes=[pltpu.VMEM((_K, _TILE), jnp.float32)],
    )
    return pl.pallas_call(
        _moe_body,
        grid_spec=grid_spec,
        out_shape=jax.ShapeDtypeStruct((_B, _C2, _NF), jnp.float32),
        compiler_params=pltpu.CompilerParams(
            dimension_semantics=("arbitrary",)),
    )(top_idx, gates, xs2, wt, wt, prm, prm)


def kernel(x, w_gate, proj_w, proj_b, ln_w, ln_b):
    xr = x.reshape(_B, _C1, _H56, 4, _H56, 4)
    xr = xr.transpose(0, 1, 3, 5, 2, 4)
    xs2 = xr.reshape(_B, 48, _NPAD)

    wp = jnp.pad(proj_w, ((0, 0), (0, 0), (0, 0), (0, 1), (0, 1)))
    wt = wp.reshape(_E, _C2, _C1, 2, 4, 2, 4)
    wt = wt.transpose(0, 1, 3, 5, 2, 4, 6).reshape(_E, _C2, _K)

    prm = jnp.stack([proj_b, ln_w, ln_b], axis=-1)

    logits = _pool_call(x.reshape(_B, _C1 * _HW, _HW), w_gate.T)
    top_idx, gates = _route_call()(logits)
    yf = _moe_call(top_idx, gates, xs2, wt, prm)
    return yf.reshape(_B, _C2, _HO, _H56)[..., :_HO]
